# baseline (speedup 1.0000x reference)
import functools

import jax
import jax.numpy as jnp
from jax import lax
from jax.experimental import pallas as pl
from jax.experimental.pallas import tpu as pltpu
from jax.experimental.pallas import tpu_sc as plsc

VOCAB = 1000000
EMB = 32
OUT = 5
B = 4096
L = 200

PW = 16
PACK = 128 // PW
PROWS = VOCAB // PACK
W128 = 128

NC = 2
NS = 16
NW = NC * NS
BAGS_PER_W = B // NW
CHUNK_BAGS = 2
TOK_PER_CHUNK = CHUNK_BAGS * L
STREAM_W = 80
STREAMS_PER_CHUNK = TOK_PER_CHUNK // STREAM_W
CHUNKS = BAGS_PER_W // CHUNK_BAGS
PAIR_TOK = 2 * TOK_PER_CHUNK
PAIRS = CHUNKS // 2
TOK_PER_W = BAGS_PER_W * L

PROJ_BLK = 5000


def _tc_project(embed, wt16):
    def body(*refs):
        e_refs, w_ref, o_ref = refs[:PACK], refs[PACK], refs[PACK + 1]
        for j in range(PACK):
            y = jnp.dot(e_refs[j][:], w_ref[:],
                        precision=lax.Precision.HIGHEST,
                        preferred_element_type=jnp.float32)
            o_ref[:, j * PW:(j + 1) * PW] = y

    def make_spec(j):
        return pl.BlockSpec((PROJ_BLK, EMB),
                            lambda i, j=j: (j * (PROWS // PROJ_BLK) + i, 0))

    return pl.pallas_call(
        body,
        grid=(PROWS // PROJ_BLK,),
        in_specs=[make_spec(j) for j in range(PACK)]
        + [pl.BlockSpec((EMB, PW), lambda i: (0, 0))],
        out_specs=pl.BlockSpec((PROJ_BLK, W128), lambda i: (i, 0)),
        out_shape=jax.ShapeDtypeStruct((PROWS, W128), jnp.float32),
    )(*([embed] * PACK), wt16)


def _sc_pool(ptable, data1d):
    mesh = plsc.VectorSubcoreMesh(core_axis_name="c", subcore_axis_name="s")

    @functools.partial(
        pl.kernel,
        mesh=mesh,
        out_type=jax.ShapeDtypeStruct((B // 8, W128), jnp.float32),
        scratch_types=[
            pltpu.VMEM((PAIR_TOK + 16,), jnp.int32),
            pltpu.VMEM((PAIR_TOK,), jnp.int32),
            pltpu.VMEM((PAIR_TOK + 16,), jnp.int32),
            pltpu.VMEM((TOK_PER_CHUNK, W128), jnp.float32),
            pltpu.VMEM((TOK_PER_CHUNK, W128), jnp.float32),
            pltpu.VMEM((1, W128), jnp.float32),
            pltpu.SemaphoreType.DMA,
            pltpu.SemaphoreType.DMA,
            pltpu.SemaphoreType.DMA,
        ],
    )
    def pool(table_hbm, data_hbm, out_hbm, idx_v, row8_v, sub_v, rows_a,
             rows_b, stage_v, isem, sem_a, sem_b):
        wid = lax.axis_index("s") * NC + lax.axis_index("c")
        tok0 = wid * TOK_PER_W

        def load_pair(k):
            pltpu.async_copy(
                data_hbm.at[pl.ds(tok0 + k * PAIR_TOK, PAIR_TOK)],
                idx_v.at[pl.ds(0, PAIR_TOK)],
                isem,
            ).wait()
            zero16 = jnp.zeros((16,), jnp.int32)
            for g in range(PAIR_TOK // 16):
                v = idx_v[pl.ds(g * 16, 16)]
                off = zero16
                a = v >= 4 * PROWS
                v = jnp.where(a, v - 4 * PROWS, v)
                off = jnp.where(a, off + 64, off)
                bq = v >= 2 * PROWS
                v = jnp.where(bq, v - 2 * PROWS, v)
                off = jnp.where(bq, off + 32, off)
                c = v >= PROWS
                row8_v[pl.ds(g * 16, 16)] = jnp.where(c, v - PROWS, v)
                sub_v[pl.ds(g * 16, 16)] = jnp.where(c, off + 16, off)

        def fire(half, rows_v, sem):
            for j in range(STREAMS_PER_CHUNK):
                pltpu.async_copy(
                    table_hbm.at[
                        row8_v.at[pl.ds((half * STREAMS_PER_CHUNK + j) * STREAM_W,
                                        STREAM_W)]
                    ],
                    rows_v.at[pl.ds(j * STREAM_W, STREAM_W)],
                    sem,
                )

        def drain(rows_v, sem):
            pltpu.make_async_copy(
                table_hbm.at[pl.ds(0, TOK_PER_CHUNK)], rows_v, sem
            ).wait()

        def acc(kk, half, rows_v):
            for i in range(CHUNK_BAGS):
                lane0 = (kk * 4 + half * CHUNK_BAGS + i) * PW

                def row_body(r, a0):
                    tok = i * L + r * 8
                    svec = sub_v[pl.ds(half * TOK_PER_CHUNK + tok, 16)]
                    for u in range(8):
                        s = svec[u]
                        a0 = a0 + rows_v[tok + u, pl.ds(s, 16)]
                    return a0

                zero = jnp.zeros((16,), jnp.float32)
                a0 = lax.fori_loop(0, L // 8, row_body, zero)
                stage_v[0, pl.ds(lane0, 16)] = a0

        def quad_body(m, carry):
            for kk in range(2):
                k = m * 2 + kk
                load_pair(k)
                fire(0, rows_a, sem_a)
                fire(1, rows_b, sem_b)
                drain(rows_a, sem_a)
                acc(kk, 0, rows_a)
                drain(rows_b, sem_b)
                acc(kk, 1, rows_b)
            pltpu.sync_copy(
                stage_v, out_hbm.at[pl.ds(wid * (PAIRS // 2) + m, 1)]
            )
            return carry

        lax.fori_loop(0, PAIRS // 2, quad_body, 0)

    return pool(ptable, data1d)


def _tc_head(pooled16, inv_len, b16):
    BLK = 512

    def body(p_ref, il_ref, b_ref, o_ref):
        o_ref[:] = p_ref[:] * il_ref[:] + b_ref[:]

    return pl.pallas_call(
        body,
        grid=(B // BLK,),
        in_specs=[
            pl.BlockSpec((BLK, PW), lambda i: (i, 0)),
            pl.BlockSpec((BLK, 1), lambda i: (i, 0)),
            pl.BlockSpec((1, PW), lambda i: (0, 0)),
        ],
        out_specs=pl.BlockSpec((BLK, PW), lambda i: (i, 0)),
        out_shape=jax.ShapeDtypeStruct((B, PW), jnp.float32),
    )(pooled16, inv_len, b16)


def kernel(data, length, embed, W, b):
    wt16 = jnp.zeros((EMB, PW), jnp.float32).at[:, :OUT].set(W.T)
    ptable = _tc_project(embed, wt16)
    pooled8 = _sc_pool(ptable, data.reshape(B * L))
    pooled16 = pooled8.reshape(B, PW)
    inv_len = (1.0 / length.astype(jnp.float32)).reshape(B, 1)
    b16 = jnp.zeros((1, PW), jnp.float32).at[0, :OUT].set(b)
    out16 = _tc_head(pooled16, inv_len, b16)
    return out16[:, :OUT]


# speedup vs baseline: 1.8474x; 1.8474x over previous
import functools

import jax
import jax.numpy as jnp
from jax import lax
from jax.experimental import pallas as pl
from jax.experimental.pallas import tpu as pltpu
from jax.experimental.pallas import tpu_sc as plsc

VOCAB = 1000000
EMB = 32
OUT = 5
B = 4096
L = 200

PW = 16
PACK = 128 // PW
SUB = 4096
GSTEP = PACK * SUB
NSTEP = (VOCAB + GSTEP - 1) // GSTEP
PROWS = NSTEP * SUB
W128 = 128

NC = 2
NS = 16
NW = NC * NS
BAGS_PER_W = B // NW
CHUNK_BAGS = 2
TOK_PER_CHUNK = CHUNK_BAGS * L
STREAM_W = 80
STREAMS_PER_CHUNK = TOK_PER_CHUNK // STREAM_W
CHUNKS = BAGS_PER_W // CHUNK_BAGS
PAIR_TOK = 2 * TOK_PER_CHUNK
PAIRS = CHUNKS // 2
TOK_PER_W = BAGS_PER_W * L


def _tc_project(embed, wbig):
    def body(e_ref, w_ref, o_ref):
        x = jnp.concatenate(
            [e_ref[j * SUB:(j + 1) * SUB, :] for j in range(PACK)], axis=1)
        o_ref[:] = jnp.dot(x, w_ref[:], preferred_element_type=jnp.float32)

    return pl.pallas_call(
        body,
        grid=(NSTEP,),
        in_specs=[
            pl.BlockSpec((GSTEP, EMB), lambda i: (i, 0)),
            pl.BlockSpec((PACK * EMB, W128), lambda i: (0, 0)),
        ],
        out_specs=pl.BlockSpec((SUB, W128), lambda i: (i, 0)),
        out_shape=jax.ShapeDtypeStruct((PROWS, W128), jnp.float32),
    )(embed, wbig)


def _sc_pool(ptable, data1d):
    mesh = plsc.VectorSubcoreMesh(core_axis_name="c", subcore_axis_name="s")

    @functools.partial(
        pl.kernel,
        mesh=mesh,
        out_type=jax.ShapeDtypeStruct((B // 8, W128), jnp.float32),
        scratch_types=[
            pltpu.VMEM((PAIR_TOK + 16,), jnp.int32),
            pltpu.VMEM((PAIR_TOK,), jnp.int32),
            pltpu.VMEM((PAIR_TOK + 16,), jnp.int32),
            pltpu.VMEM((TOK_PER_CHUNK, W128), jnp.float32),
            pltpu.VMEM((TOK_PER_CHUNK, W128), jnp.float32),
            pltpu.VMEM((1, W128), jnp.float32),
            pltpu.SemaphoreType.DMA,
            pltpu.SemaphoreType.DMA,
            pltpu.SemaphoreType.DMA,
        ],
    )
    def pool(table_hbm, data_hbm, out_hbm, idx_v, row8_v, sub_v, rows_a,
             rows_b, stage_v, isem, sem_a, sem_b):
        wid = lax.axis_index("s") * NC + lax.axis_index("c")
        tok0 = wid * TOK_PER_W

        def load_pair(k):
            pltpu.async_copy(
                data_hbm.at[pl.ds(tok0 + k * PAIR_TOK, PAIR_TOK)],
                idx_v.at[pl.ds(0, PAIR_TOK)],
                isem,
            ).wait()
            for g in range(PAIR_TOK // 16):
                v = idx_v[pl.ds(g * 16, 16)]
                step = lax.shift_right_logical(v, 15)
                row8_v[pl.ds(g * 16, 16)] = step * SUB + (v & (SUB - 1))
                sub_v[pl.ds(g * 16, 16)] = (
                    lax.shift_right_logical(v, 12) & 7) * PW

        def fire(half, rows_v, sem):
            for j in range(STREAMS_PER_CHUNK):
                pltpu.async_copy(
                    table_hbm.at[
                        row8_v.at[pl.ds((half * STREAMS_PER_CHUNK + j) * STREAM_W,
                                        STREAM_W)]
                    ],
                    rows_v.at[pl.ds(j * STREAM_W, STREAM_W)],
                    sem,
                )

        def drain(rows_v, sem):
            pltpu.make_async_copy(
                table_hbm.at[pl.ds(0, TOK_PER_CHUNK)], rows_v, sem
            ).wait()

        def acc(kk, half, rows_v):
            for i in range(CHUNK_BAGS):
                lane0 = (kk * 4 + half * CHUNK_BAGS + i) * PW

                def row_body(r, a0):
                    tok = i * L + r * 8
                    svec = sub_v[pl.ds(half * TOK_PER_CHUNK + tok, 16)]
                    for u in range(8):
                        s = svec[u]
                        a0 = a0 + rows_v[tok + u, pl.ds(s, 16)]
                    return a0

                zero = jnp.zeros((16,), jnp.float32)
                a0 = lax.fori_loop(0, L // 8, row_body, zero)
                stage_v[0, pl.ds(lane0, 16)] = a0

        def quad_body(m, carry):
            for kk in range(2):
                k = m * 2 + kk
                load_pair(k)
                fire(0, rows_a, sem_a)
                fire(1, rows_b, sem_b)
                drain(rows_a, sem_a)
                acc(kk, 0, rows_a)
                drain(rows_b, sem_b)
                acc(kk, 1, rows_b)
            pltpu.sync_copy(
                stage_v, out_hbm.at[pl.ds(wid * (PAIRS // 2) + m, 1)]
            )
            return carry

        lax.fori_loop(0, PAIRS // 2, quad_body, 0)

    return pool(ptable, data1d)


def _tc_head(pooled16, inv_len, b16):
    BLK = 512

    def body(p_ref, il_ref, b_ref, o_ref):
        o_ref[:] = p_ref[:] * il_ref[:] + b_ref[:]

    return pl.pallas_call(
        body,
        grid=(B // BLK,),
        in_specs=[
            pl.BlockSpec((BLK, PW), lambda i: (i, 0)),
            pl.BlockSpec((BLK, 1), lambda i: (i, 0)),
            pl.BlockSpec((1, PW), lambda i: (0, 0)),
        ],
        out_specs=pl.BlockSpec((BLK, PW), lambda i: (i, 0)),
        out_shape=jax.ShapeDtypeStruct((B, PW), jnp.float32),
    )(pooled16, inv_len, b16)


def kernel(data, length, embed, W, b):
    wbig = jnp.zeros((PACK * EMB, W128), jnp.float32)
    for j in range(PACK):
        wbig = wbig.at[j * EMB:(j + 1) * EMB, j * PW:j * PW + OUT].set(W.T)
    ptable = _tc_project(embed, wbig)
    pooled8 = _sc_pool(ptable, data.reshape(B * L))
    pooled16 = pooled8.reshape(B, PW)
    inv_len = (1.0 / length.astype(jnp.float32)).reshape(B, 1)
    b16 = jnp.zeros((1, PW), jnp.float32).at[0, :OUT].set(b)
    out16 = _tc_head(pooled16, inv_len, b16)
    return out16[:, :OUT]
